# baseline (speedup 1.0000x reference)
import functools

import jax
import jax.numpy as jnp
from jax import lax
from jax.experimental import pallas as pl
from jax.experimental.pallas import tpu as pltpu
from jax.experimental.pallas import tpu_sc as plsc

VOCAB = 100000
EMBED = 300
CLASSES = 10
B = 4096
L = 50

NC = 2
NS = 16
NW = NC * NS
ROWS_PER_W = B // NW

LANES = 16
MM_BLOCK = 6400
VOCAB_PAD = 100032


def _mm_body(t_ref, w_ref, o_ref):
    res = jax.lax.dot_general(
        t_ref[...], w_ref[...], (((0,), (0,)), ((), ())),
        preferred_element_type=jnp.float32)
    res3 = res.reshape(MM_BLOCK // 8, 8, LANES)
    for s in range(8):
        o_ref[:, s * LANES:(s + 1) * LANES] = res3[:, s, :]


def _fold_table(table_t, wp):
    grid = (pl.cdiv(VOCAB, MM_BLOCK),)
    packed = pl.pallas_call(
        _mm_body,
        grid=grid,
        in_specs=[
            pl.BlockSpec((EMBED, MM_BLOCK), lambda i: (0, i)),
            pl.BlockSpec((EMBED, LANES), lambda i: (0, 0)),
        ],
        out_specs=pl.BlockSpec((MM_BLOCK // 8, 8 * LANES), lambda i: (i, 0)),
        out_shape=jax.ShapeDtypeStruct((VOCAB_PAD // 8, 8 * LANES), jnp.float32),
    )(table_t, wp)
    return packed.reshape(VOCAB_PAD, LANES)


TOK_PER_W = ROWS_PER_W * L
GATHER_CHUNK = 128
N_CHUNKS = TOK_PER_W // GATHER_CHUNK


def _sc_body(r_hbm, idx_hbm, bias_hbm, out_hbm, idx_v, g_v, acc_v, bias_v, sem):
    wid = lax.axis_index("s") * NC + lax.axis_index("c")
    pltpu.sync_copy(idx_hbm.at[wid], idx_v)
    pltpu.sync_copy(bias_hbm, bias_v)

    def fire_c(c, _):
        pltpu.async_copy(r_hbm.at[idx_v.at[pl.ds(c * GATHER_CHUNK, GATHER_CHUNK)]],
                         g_v.at[pl.ds(c * GATHER_CHUNK, GATHER_CHUNK)], sem)
        return 0
    lax.fori_loop(0, N_CHUNKS, fire_c, 0)

    def drain_c(c, _):
        pltpu.make_async_copy(
            r_hbm.at[idx_v.at[pl.ds(c * GATHER_CHUNK, GATHER_CHUNK)]],
            g_v.at[pl.ds(c * GATHER_CHUNK, GATHER_CHUNK)], sem).wait()
        return 0
    lax.fori_loop(0, N_CHUNKS, drain_c, 0)

    bias_vec = bias_v[...]

    def reduce_r(r, _):
        base = r * L
        accs = [bias_vec, jnp.zeros((LANES,), jnp.float32),
                jnp.zeros((LANES,), jnp.float32), jnp.zeros((LANES,), jnp.float32)]
        for j in range(L):
            accs[j % 4] = accs[j % 4] + g_v[base + j]
        acc_v[r] = (accs[0] + accs[1]) + (accs[2] + accs[3])
        return 0
    lax.fori_loop(0, ROWS_PER_W, reduce_r, 0)

    pltpu.sync_copy(acc_v, out_hbm.at[pl.ds(wid * ROWS_PER_W, ROWS_PER_W)])


_sc_pool = functools.partial(
    pl.kernel,
    out_type=jax.ShapeDtypeStruct((B, LANES), jnp.float32),
    mesh=plsc.VectorSubcoreMesh(core_axis_name="c", subcore_axis_name="s"),
    compiler_params=pltpu.CompilerParams(use_tc_tiling_on_sc=False),
    scratch_types=[
        pltpu.VMEM((TOK_PER_W,), jnp.int32),
        pltpu.VMEM((TOK_PER_W, LANES), jnp.float32),
        pltpu.VMEM((ROWS_PER_W, LANES), jnp.float32),
        pltpu.VMEM((LANES,), jnp.float32),
        pltpu.SemaphoreType.DMA,
    ],
)(_sc_body)


def kernel(x, table, W, b):
    x = x.astype(jnp.int32)
    wp = jnp.pad(W, ((0, 0), (0, LANES - CLASSES))) * (1.0 / L)
    bp = jnp.pad(b, (0, LANES - CLASSES))
    r = _fold_table(table.T, wp)
    idx = x.reshape(NW, TOK_PER_W)
    out = _sc_pool(r, idx, bp)
    return out[:, :CLASSES]


# speedup vs baseline: 1.0275x; 1.0275x over previous
import functools

import jax
import jax.numpy as jnp
from jax import lax
from jax.experimental import pallas as pl
from jax.experimental.pallas import tpu as pltpu
from jax.experimental.pallas import tpu_sc as plsc

VOCAB = 100000
EMBED = 300
CLASSES = 10
B = 4096
L = 50

NC = 2
NS = 16
NW = NC * NS
ROWS_PER_W = B // NW

LANES = 16
MM_BLOCK = 12800
VOCAB_PAD = 100032


def _mm_body(t_ref, w_ref, o_ref):
    res = jax.lax.dot_general(
        t_ref[...], w_ref[...], (((0,), (0,)), ((), ())),
        preferred_element_type=jnp.float32)
    res3 = res.reshape(MM_BLOCK // 8, 8, LANES)
    for s in range(8):
        o_ref[:, s * LANES:(s + 1) * LANES] = res3[:, s, :]


def _fold_table(table_t, wp):
    grid = (pl.cdiv(VOCAB, MM_BLOCK),)
    packed = pl.pallas_call(
        _mm_body,
        grid=grid,
        in_specs=[
            pl.BlockSpec((EMBED, MM_BLOCK), lambda i: (0, i)),
            pl.BlockSpec((EMBED, LANES), lambda i: (0, 0)),
        ],
        out_specs=pl.BlockSpec((MM_BLOCK // 8, 8 * LANES), lambda i: (i, 0)),
        out_shape=jax.ShapeDtypeStruct((VOCAB_PAD // 8, 8 * LANES), jnp.float32),
    )(table_t, wp)
    return packed.reshape(VOCAB_PAD, LANES)


TOK_PER_W = ROWS_PER_W * L
GATHER_CHUNK = 128
N_CHUNKS = TOK_PER_W // GATHER_CHUNK


def _sc_body(r_hbm, idx_hbm, bias_hbm, out_hbm, idx_v, g_v, acc_v, bias_v, sem):
    wid = lax.axis_index("s") * NC + lax.axis_index("c")
    pltpu.sync_copy(idx_hbm.at[wid], idx_v)
    pltpu.sync_copy(bias_hbm, bias_v)

    def fire_c(c, _):
        pltpu.async_copy(r_hbm.at[idx_v.at[pl.ds(c * GATHER_CHUNK, GATHER_CHUNK)]],
                         g_v.at[pl.ds(c * GATHER_CHUNK, GATHER_CHUNK)], sem)
        return 0
    lax.fori_loop(0, N_CHUNKS, fire_c, 0)

    def drain_c(c, _):
        pltpu.make_async_copy(
            r_hbm.at[idx_v.at[pl.ds(c * GATHER_CHUNK, GATHER_CHUNK)]],
            g_v.at[pl.ds(c * GATHER_CHUNK, GATHER_CHUNK)], sem).wait()
        return 0

    bias_vec = bias_v[...]

    def reduce_r(r, _):
        base = r * L
        accs = [bias_vec, jnp.zeros((LANES,), jnp.float32),
                jnp.zeros((LANES,), jnp.float32), jnp.zeros((LANES,), jnp.float32)]
        for j in range(L):
            accs[j % 4] = accs[j % 4] + g_v[base + j]
        acc_v[r] = (accs[0] + accs[1]) + (accs[2] + accs[3])
        return 0

    lax.fori_loop(0, N_CHUNKS // 2, drain_c, 0)
    lax.fori_loop(0, ROWS_PER_W // 2, reduce_r, 0)
    lax.fori_loop(N_CHUNKS // 2, N_CHUNKS, drain_c, 0)
    lax.fori_loop(ROWS_PER_W // 2, ROWS_PER_W, reduce_r, 0)

    pltpu.sync_copy(acc_v, out_hbm.at[pl.ds(wid * ROWS_PER_W, ROWS_PER_W)])


_sc_pool = functools.partial(
    pl.kernel,
    out_type=jax.ShapeDtypeStruct((B, LANES), jnp.float32),
    mesh=plsc.VectorSubcoreMesh(core_axis_name="c", subcore_axis_name="s"),
    compiler_params=pltpu.CompilerParams(use_tc_tiling_on_sc=False),
    scratch_types=[
        pltpu.VMEM((TOK_PER_W,), jnp.int32),
        pltpu.VMEM((TOK_PER_W, LANES), jnp.float32),
        pltpu.VMEM((ROWS_PER_W, LANES), jnp.float32),
        pltpu.VMEM((LANES,), jnp.float32),
        pltpu.SemaphoreType.DMA,
    ],
)(_sc_body)


def kernel(x, table, W, b):
    x = x.astype(jnp.int32)
    wp = jnp.pad(W, ((0, 0), (0, LANES - CLASSES))) * (1.0 / L)
    bp = jnp.pad(b, (0, LANES - CLASSES))
    r = _fold_table(table.T, wp)
    idx = x.reshape(NW, TOK_PER_W)
    out = _sc_pool(r, idx, bp)
    return out[:, :CLASSES]
